# Initial kernel scaffold; baseline (speedup 1.0000x reference)
import jax
import jax.numpy as jnp
from jax.experimental import pallas as pl


def kernel(sources):
    raise NotImplementedError("write your pallas kernel here")


# speedup vs baseline: 3.4328x; 3.4328x over previous
import functools

import jax
import jax.numpy as jnp
import numpy as np
from jax.experimental import pallas as pl
from jax.experimental.pallas import tpu as pltpu

_B = 64
_T = 160000


def _compute_gather_idx() -> np.ndarray:
    pkey = jax.random.key(42)
    perm = np.asarray(jnp.argsort(jax.random.uniform(pkey, (_B,))))
    return np.concatenate([perm, _B + np.arange(_B)]).astype(np.int32)


_GATHER_IDX = _compute_gather_idx()


def _copy_body(g_ref, src_ref, out_ref):
    out_ref[...] = src_ref[...]


def kernel(sources):
    flat = sources.reshape(2 * _B, 1, _T)
    out = pl.pallas_call(
        _copy_body,
        grid_spec=pltpu.PrefetchScalarGridSpec(
            num_scalar_prefetch=1,
            grid=(2 * _B,),
            in_specs=[pl.BlockSpec((1, 1, _T), lambda i, g: (g[i], 0, 0))],
            out_specs=pl.BlockSpec((1, 1, _T), lambda i, g: (i, 0, 0)),
        ),
        out_shape=jax.ShapeDtypeStruct((2 * _B, 1, _T), jnp.float32),
    )(jnp.asarray(_GATHER_IDX), flat)
    return out.reshape(2, _B, 1, _T)
